# baseline (speedup 1.0000x reference)
import functools

import numpy as np
import jax
import jax.numpy as jnp
from jax import lax
from jax.experimental import pallas as pl
from jax.experimental.pallas import tpu as pltpu
from jax.experimental.pallas import tpu_sc as plsc

_ROWS = 16384
_COLS = 16
_NQ = 4
_NOUT = _NQ // 2

_NC = 2
_NS = 16
_NW = _NC * _NS
_RPW = _ROWS // _NW
_BLKS = _RPW // 16


def _sign_table():
    basis = np.arange(2 ** _NQ)
    signs = np.zeros((_NOUT, _COLS), dtype=np.float64)
    for i in range(_NOUT):
        ind = i * 2
        hi_bit = (basis >> (_NQ - 1 - ind)) & 1
        lo_bit = (basis >> (_NQ - 2 - ind)) & 1
        signs[i] = np.where(hi_bit == lo_bit, 1.0, -1.0)
    return signs


_SIGNS = _sign_table()


def _signed_tree_sum(cols, signs):
    terms = list(cols)
    sgn = list(signs)
    while len(terms) > 1:
        nxt_t, nxt_s = [], []
        for k in range(0, len(terms), 2):
            a, sa = terms[k], sgn[k]
            b, sb = terms[k + 1], sgn[k + 1]
            if sa == sb:
                nxt_t.append(a + b)
                nxt_s.append(sa)
            else:
                nxt_t.append(a - b if sa > 0 else b - a)
                nxt_s.append(1.0)
        terms, sgn = nxt_t, nxt_s
    return terms[0] if sgn[0] > 0 else -terms[0]


@functools.cache
def _build_decode_sc():
    mesh = plsc.VectorSubcoreMesh(
        core_axis_name="c", subcore_axis_name="s", num_cores=_NC
    )

    @functools.partial(
        pl.kernel,
        out_type=jax.ShapeDtypeStruct((_NOUT, _ROWS), jnp.float32),
        mesh=mesh,
        scratch_types=[
            pltpu.VMEM((_COLS, _RPW), jnp.float32),
            pltpu.VMEM((_RPW,), jnp.float32),
            pltpu.VMEM((_RPW,), jnp.float32),
        ],
        compiler_params=pltpu.CompilerParams(
            needs_layout_passes=False,
            disable_bounds_checks=True,
            disable_semaphore_checks=True,
            skip_device_barrier=True,
        ),
    )
    def _decode_sc(xt_hbm, out_hbm, x_v, o0_v, o1_v):
        wid = lax.axis_index("s") * _NC + lax.axis_index("c")
        base = wid * _RPW
        pltpu.sync_copy(xt_hbm.at[:, pl.ds(base, _RPW)], x_v)

        @plsc.parallel_loop(0, _RPW, 16, unroll=2)
        def block(rr):
            cols = [x_v[j, pl.ds(rr, 16)] for j in range(_COLS)]
            o0_v[pl.ds(rr, 16)] = _signed_tree_sum(cols, _SIGNS[0])
            o1_v[pl.ds(rr, 16)] = _signed_tree_sum(cols, _SIGNS[1])

        pltpu.sync_copy(o0_v, out_hbm.at[0, pl.ds(base, _RPW)])
        pltpu.sync_copy(o1_v, out_hbm.at[1, pl.ds(base, _RPW)])

    return _decode_sc


def kernel(input):
    out_t = _build_decode_sc()(input.T)
    return out_t.T


# speedup vs baseline: 1.0574x; 1.0574x over previous
import functools

import numpy as np
import jax
import jax.numpy as jnp
from jax import lax
from jax.experimental import pallas as pl
from jax.experimental.pallas import tpu as pltpu
from jax.experimental.pallas import tpu_sc as plsc

_ROWS = 16384
_COLS = 16
_NQ = 4
_NOUT = _NQ // 2

_NC = 1
_NS = 16
_NW = _NC * _NS
_RPW = _ROWS // _NW
_BLKS = _RPW // 16


def _sign_table():
    basis = np.arange(2 ** _NQ)
    signs = np.zeros((_NOUT, _COLS), dtype=np.float64)
    for i in range(_NOUT):
        ind = i * 2
        hi_bit = (basis >> (_NQ - 1 - ind)) & 1
        lo_bit = (basis >> (_NQ - 2 - ind)) & 1
        signs[i] = np.where(hi_bit == lo_bit, 1.0, -1.0)
    return signs


_SIGNS = _sign_table()


def _signed_tree_sum(cols, signs):
    terms = list(cols)
    sgn = list(signs)
    while len(terms) > 1:
        nxt_t, nxt_s = [], []
        for k in range(0, len(terms), 2):
            a, sa = terms[k], sgn[k]
            b, sb = terms[k + 1], sgn[k + 1]
            if sa == sb:
                nxt_t.append(a + b)
                nxt_s.append(sa)
            else:
                nxt_t.append(a - b if sa > 0 else b - a)
                nxt_s.append(1.0)
        terms, sgn = nxt_t, nxt_s
    return terms[0] if sgn[0] > 0 else -terms[0]


@functools.cache
def _build_decode_sc():
    mesh = plsc.VectorSubcoreMesh(
        core_axis_name="c", subcore_axis_name="s", num_cores=_NC
    )

    @functools.partial(
        pl.kernel,
        out_type=jax.ShapeDtypeStruct((_NOUT, _ROWS), jnp.float32),
        mesh=mesh,
        scratch_types=[
            pltpu.VMEM((_COLS, _RPW), jnp.float32),
            pltpu.VMEM((_RPW,), jnp.float32),
            pltpu.VMEM((_RPW,), jnp.float32),
            pltpu.SemaphoreType.DMA,
            pltpu.SemaphoreType.DMA,
        ],
        compiler_params=pltpu.CompilerParams(
            needs_layout_passes=False,
            disable_bounds_checks=True,
            disable_semaphore_checks=True,
            skip_device_barrier=True,
        ),
    )
    def _decode_sc(xt_hbm, out_hbm, x_v, o0_v, o1_v, sem_a, sem_b):
        wid = lax.axis_index("s") * _NC + lax.axis_index("c")
        base = wid * _RPW
        half = _RPW // 2
        cp_a = pltpu.async_copy(
            xt_hbm.at[:, pl.ds(base, half)], x_v.at[:, pl.ds(0, half)], sem_a
        )
        cp_b = pltpu.async_copy(
            xt_hbm.at[:, pl.ds(base + half, half)],
            x_v.at[:, pl.ds(half, half)],
            sem_b,
        )
        cp_a.wait()

        @plsc.parallel_loop(0, half, 16, unroll=2)
        def block_a(rr):
            cols = [x_v[j, pl.ds(rr, 16)] for j in range(_COLS)]
            o0_v[pl.ds(rr, 16)] = _signed_tree_sum(cols, _SIGNS[0])
            o1_v[pl.ds(rr, 16)] = _signed_tree_sum(cols, _SIGNS[1])

        cp_b.wait()

        @plsc.parallel_loop(half, _RPW, 16, unroll=2)
        def block_b(rr):
            cols = [x_v[j, pl.ds(rr, 16)] for j in range(_COLS)]
            o0_v[pl.ds(rr, 16)] = _signed_tree_sum(cols, _SIGNS[0])
            o1_v[pl.ds(rr, 16)] = _signed_tree_sum(cols, _SIGNS[1])

        pltpu.sync_copy(o0_v, out_hbm.at[0, pl.ds(base, _RPW)])
        pltpu.sync_copy(o1_v, out_hbm.at[1, pl.ds(base, _RPW)])

    return _decode_sc


def kernel(input):
    out_t = _build_decode_sc()(input.T)
    return out_t.T


# speedup vs baseline: 1.0780x; 1.0195x over previous
import functools

import numpy as np
import jax
import jax.numpy as jnp
from jax import lax
from jax.experimental import pallas as pl
from jax.experimental.pallas import tpu as pltpu
from jax.experimental.pallas import tpu_sc as plsc

_ROWS = 16384
_COLS = 16
_NQ = 4
_NOUT = _NQ // 2

_NC = 1
_NS = 16
_NW = _NC * _NS
_RPW = _ROWS // _NW
_BLKS = _RPW // 16


def _sign_table():
    basis = np.arange(2 ** _NQ)
    signs = np.zeros((_NOUT, _COLS), dtype=np.float64)
    for i in range(_NOUT):
        ind = i * 2
        hi_bit = (basis >> (_NQ - 1 - ind)) & 1
        lo_bit = (basis >> (_NQ - 2 - ind)) & 1
        signs[i] = np.where(hi_bit == lo_bit, 1.0, -1.0)
    return signs


_SIGNS = _sign_table()


def _signed_tree_sum(cols, signs):
    terms = list(cols)
    sgn = list(signs)
    while len(terms) > 1:
        nxt_t, nxt_s = [], []
        for k in range(0, len(terms), 2):
            a, sa = terms[k], sgn[k]
            b, sb = terms[k + 1], sgn[k + 1]
            if sa == sb:
                nxt_t.append(a + b)
                nxt_s.append(sa)
            else:
                nxt_t.append(a - b if sa > 0 else b - a)
                nxt_s.append(1.0)
        terms, sgn = nxt_t, nxt_s
    return terms[0] if sgn[0] > 0 else -terms[0]


@functools.cache
def _build_decode_sc():
    mesh = plsc.VectorSubcoreMesh(
        core_axis_name="c", subcore_axis_name="s", num_cores=_NC
    )

    @functools.partial(
        pl.kernel,
        out_type=jax.ShapeDtypeStruct((_NOUT, _ROWS), jnp.float32),
        mesh=mesh,
        scratch_types=[
            pltpu.VMEM((_COLS, _RPW), jnp.float32),
            pltpu.VMEM((_RPW,), jnp.float32),
            pltpu.VMEM((_RPW,), jnp.float32),
        ],
        compiler_params=pltpu.CompilerParams(
            needs_layout_passes=False,
            disable_bounds_checks=True,
            disable_semaphore_checks=True,
            skip_device_barrier=True,
        ),
    )
    def _decode_sc(xt_hbm, out_hbm, x_v, o0_v, o1_v):
        wid = lax.axis_index("s") * _NC + lax.axis_index("c")
        base = wid * _RPW
        pltpu.sync_copy(xt_hbm.at[:, pl.ds(base, _RPW)], x_v)

        @plsc.parallel_loop(0, _RPW, 16, unroll=2)
        def block(rr):
            cols = [x_v[j, pl.ds(rr, 16)] for j in range(_COLS)]
            o0_v[pl.ds(rr, 16)] = _signed_tree_sum(cols, _SIGNS[0])
            o1_v[pl.ds(rr, 16)] = _signed_tree_sum(cols, _SIGNS[1])

        pltpu.sync_copy(o0_v, out_hbm.at[0, pl.ds(base, _RPW)])
        pltpu.sync_copy(o1_v, out_hbm.at[1, pl.ds(base, _RPW)])

    return _decode_sc


def kernel(input):
    out_t = _build_decode_sc()(input.T)
    return out_t.T
